# Initial kernel scaffold; baseline (speedup 1.0000x reference)
import jax
import jax.numpy as jnp
from jax.experimental import pallas as pl


def kernel(x_context, x_target, params):
    raise NotImplementedError("write your pallas kernel here")


# speedup vs baseline: 1.7975x; 1.7975x over previous
import functools

import jax
import jax.numpy as jnp
from jax import lax
from jax.experimental import pallas as pl
from jax.experimental.pallas import tpu as pltpu

DIM = 1024
LAYERS = 3
NUM_EXPERTS = 4
TOP_K = 2
RANK = 16
TB = 512


def _layernorm(y):
    mu = jnp.mean(y, axis=-1, keepdims=True)
    var = jnp.mean((y - mu) ** 2, axis=-1, keepdims=True)
    return (y - mu) * lax.rsqrt(var + 1e-5)


def _liquid_layers(x, win_ref, wout_ref, vec_ref):
    for l in range(LAYERS):
        bin_ = vec_ref[5 * l + 0 : 5 * l + 1, :]
        dec = vec_ref[5 * l + 1 : 5 * l + 2, :]
        bout = vec_ref[5 * l + 2 : 5 * l + 3, :]
        gam = vec_ref[5 * l + 3 : 5 * l + 4, :]
        bet = vec_ref[5 * l + 4 : 5 * l + 5, :]
        g = jax.nn.sigmoid(
            lax.dot_general(x, win_ref[l], (((1,), (1,)), ((), ())),
                            preferred_element_type=jnp.float32) + bin_)
        ns = g * (x * dec)
        out = lax.dot_general(ns, wout_ref[l], (((1,), (1,)), ((), ())),
                              preferred_element_type=jnp.float32) + bout
        x = _layernorm(out + x) * gam + bet
    return x


def _tgt_body(x_ref, win_ref, wout_ref, vec_ref, z_ref):
    z_ref[...] = _liquid_layers(x_ref[...], win_ref, wout_ref, vec_ref)


def _ctx_body(x_ref, win_ref, wout_ref, vec_ref, gw_ref, gb_ref,
              acat_ref, bcat_ref, pred_ref, probs_ref):
    z = _liquid_layers(x_ref[...], win_ref, wout_ref, vec_ref)
    logits = lax.dot_general(z, gw_ref[...], (((1,), (1,)), ((), ())),
                             preferred_element_type=jnp.float32) + gb_ref[...]
    m = jnp.max(logits, axis=-1, keepdims=True)
    e = jnp.exp(logits - m)
    probs_ref[...] = e / jnp.sum(e, axis=-1, keepdims=True)
    ii = lax.broadcasted_iota(jnp.int32, (TB, NUM_EXPERTS), 1)
    cnt = jnp.zeros((TB, NUM_EXPERTS), jnp.int32)
    for j in range(NUM_EXPERTS):
        lj = logits[:, j : j + 1]
        cnt = cnt + ((lj > logits) | ((lj == logits) & (j < ii))).astype(jnp.int32)
    mask = (cnt < TOP_K).astype(jnp.float32)
    h = lax.dot_general(z, acat_ref[...], (((1,), (1,)), ((), ())),
                        preferred_element_type=jnp.float32)
    h = 0.5 * h * (1.0 + lax.erf(h * 0.7071067811865476))
    mask64 = jnp.concatenate(
        [jnp.broadcast_to(mask[:, i : i + 1], (TB, RANK))
         for i in range(NUM_EXPERTS)], axis=1)
    pred_ref[...] = lax.dot_general(h * mask64, bcat_ref[...],
                                    (((1,), (0,)), ((), ())),
                                    preferred_element_type=jnp.float32)


def _stack_params(blocks):
    win = jnp.stack([p['win'] for p in blocks])
    wout = jnp.stack([p['wout'] for p in blocks])
    vecs = jnp.stack([v for p in blocks
                      for v in (p['bin'], p['decay'], p['bout'],
                                p['gamma'], p['beta'])])
    return win, wout, vecs


def kernel(x_context, x_target, params):
    b, s, d = x_context.shape
    n_tok = b * s
    grid = (n_tok // TB,)
    xc = x_context.reshape(n_tok, d)
    xt = x_target.reshape(n_tok, d)

    win_e, wout_e, vecs_e = _stack_params(params['encoder'])
    win_t, wout_t, vecs_t = _stack_params(params['target_encoder'])
    pred = params['predictor']
    gw = pred['gate_w']
    gb = pred['gate_b'].reshape(1, NUM_EXPERTS)
    acat = jnp.concatenate([e['A'] for e in pred['experts']], axis=0)
    bcat = jnp.concatenate([e['B'].T for e in pred['experts']], axis=0)

    tok_spec = pl.BlockSpec((TB, DIM), lambda i: (i, 0))
    full = lambda shape: pl.BlockSpec(shape, lambda i: (0,) * len(shape))

    z_target = pl.pallas_call(
        _tgt_body,
        grid=grid,
        in_specs=[tok_spec,
                  full((LAYERS, DIM, DIM)), full((LAYERS, DIM, DIM)),
                  full((5 * LAYERS, DIM))],
        out_specs=tok_spec,
        out_shape=jax.ShapeDtypeStruct((n_tok, DIM), jnp.float32),
    )(xt, win_t, wout_t, vecs_t)

    pred_z, gate_probs = pl.pallas_call(
        _ctx_body,
        grid=grid,
        in_specs=[tok_spec,
                  full((LAYERS, DIM, DIM)), full((LAYERS, DIM, DIM)),
                  full((5 * LAYERS, DIM)),
                  full((NUM_EXPERTS, DIM)), full((1, NUM_EXPERTS)),
                  full((NUM_EXPERTS * RANK, DIM)), full((NUM_EXPERTS * RANK, DIM))],
        out_specs=[tok_spec, pl.BlockSpec((TB, NUM_EXPERTS), lambda i: (i, 0))],
        out_shape=[jax.ShapeDtypeStruct((n_tok, DIM), jnp.float32),
                   jax.ShapeDtypeStruct((n_tok, NUM_EXPERTS), jnp.float32)],
    )(xc, win_e, wout_e, vecs_e, gw, gb, acat, bcat)

    return (pred_z.reshape(b, s, d),
            gate_probs.reshape(b, s, NUM_EXPERTS),
            z_target.reshape(b, s, d))


# speedup vs baseline: 1.7996x; 1.0012x over previous
import functools

import jax
import jax.numpy as jnp
from jax import lax
from jax.experimental import pallas as pl
from jax.experimental.pallas import tpu as pltpu

DIM = 1024
LAYERS = 3
NUM_EXPERTS = 4
TOP_K = 2
RANK = 16
TB = 512


def _layernorm(y):
    mu = jnp.mean(y, axis=-1, keepdims=True)
    var = jnp.mean((y - mu) ** 2, axis=-1, keepdims=True)
    return (y - mu) * lax.rsqrt(var + 1e-5)


def _liquid_layers(x, win_ref, wout_ref, vec_ref):
    for l in range(LAYERS):
        bin_ = vec_ref[5 * l + 0 : 5 * l + 1, :]
        dec = vec_ref[5 * l + 1 : 5 * l + 2, :]
        bout = vec_ref[5 * l + 2 : 5 * l + 3, :]
        gam = vec_ref[5 * l + 3 : 5 * l + 4, :]
        bet = vec_ref[5 * l + 4 : 5 * l + 5, :]
        g = jax.nn.sigmoid(
            lax.dot_general(x, win_ref[l], (((1,), (1,)), ((), ())),
                            preferred_element_type=jnp.float32) + bin_)
        ns = g * (x * dec)
        out = lax.dot_general(ns, wout_ref[l], (((1,), (1,)), ((), ())),
                              preferred_element_type=jnp.float32) + bout
        x = _layernorm(out + x) * gam + bet
    return x


def _tgt_body(x_ref, win_ref, wout_ref, vec_ref, z_ref):
    z_ref[...] = _liquid_layers(x_ref[...], win_ref, wout_ref, vec_ref)


def _ctx_body(x_ref, win_ref, wout_ref, vec_ref, gw_ref, gb_ref,
              acat_ref, bcat_ref, pred_ref, probs_ref):
    z = _liquid_layers(x_ref[...], win_ref, wout_ref, vec_ref)
    logits = lax.dot_general(z, gw_ref[...], (((1,), (1,)), ((), ())),
                             preferred_element_type=jnp.float32) + gb_ref[...]
    m = jnp.max(logits, axis=-1, keepdims=True)
    e = jnp.exp(logits - m)
    probs_ref[...] = e / jnp.sum(e, axis=-1, keepdims=True)
    ii = lax.broadcasted_iota(jnp.int32, (TB, NUM_EXPERTS), 1)
    cnt = jnp.zeros((TB, NUM_EXPERTS), jnp.int32)
    for j in range(NUM_EXPERTS):
        lj = logits[:, j : j + 1]
        cnt = cnt + ((lj > logits) | ((lj == logits) & (j < ii))).astype(jnp.int32)
    mask = (cnt < TOP_K).astype(jnp.float32)
    h = lax.dot_general(z, acat_ref[...], (((1,), (1,)), ((), ())),
                        preferred_element_type=jnp.float32)
    h = 0.5 * h * (1.0 + lax.erf(h * 0.7071067811865476))
    mask64 = jnp.concatenate(
        [jnp.broadcast_to(mask[:, i : i + 1], (TB, RANK))
         for i in range(NUM_EXPERTS)], axis=1)
    pred_ref[...] = lax.dot_general(h * mask64, bcat_ref[...],
                                    (((1,), (0,)), ((), ())),
                                    preferred_element_type=jnp.float32)


def _stack_params(blocks):
    win = jnp.stack([p['win'] for p in blocks])
    wout = jnp.stack([p['wout'] for p in blocks])
    vecs = jnp.stack([v for p in blocks
                      for v in (p['bin'], p['decay'], p['bout'],
                                p['gamma'], p['beta'])])
    return win, wout, vecs


def kernel(x_context, x_target, params):
    b, s, d = x_context.shape
    n_tok = b * s
    grid = (n_tok // TB,)
    xc = x_context.reshape(n_tok, d)
    xt = x_target.reshape(n_tok, d)

    win_e, wout_e, vecs_e = _stack_params(params['encoder'])
    win_t, wout_t, vecs_t = _stack_params(params['target_encoder'])
    pred = params['predictor']
    gw = pred['gate_w']
    gb = pred['gate_b'].reshape(1, NUM_EXPERTS)
    acat = jnp.concatenate([e['A'] for e in pred['experts']], axis=0)
    bcat = jnp.concatenate([e['B'].T for e in pred['experts']], axis=0)

    tok_spec = pl.BlockSpec((TB, DIM), lambda i: (i, 0))
    full = lambda shape: pl.BlockSpec(shape, lambda i: (0,) * len(shape))

    z_target = pl.pallas_call(
        _tgt_body,
        grid=grid,
        in_specs=[tok_spec,
                  full((LAYERS, DIM, DIM)), full((LAYERS, DIM, DIM)),
                  full((5 * LAYERS, DIM))],
        out_specs=tok_spec,
        out_shape=jax.ShapeDtypeStruct((n_tok, DIM), jnp.float32),
        compiler_params=pltpu.CompilerParams(
            dimension_semantics=("parallel",)),
    )(xt, win_t, wout_t, vecs_t)

    pred_z, gate_probs = pl.pallas_call(
        _ctx_body,
        grid=grid,
        in_specs=[tok_spec,
                  full((LAYERS, DIM, DIM)), full((LAYERS, DIM, DIM)),
                  full((5 * LAYERS, DIM)),
                  full((NUM_EXPERTS, DIM)), full((1, NUM_EXPERTS)),
                  full((NUM_EXPERTS * RANK, DIM)), full((NUM_EXPERTS * RANK, DIM))],
        out_specs=[tok_spec, pl.BlockSpec((TB, NUM_EXPERTS), lambda i: (i, 0))],
        out_shape=[jax.ShapeDtypeStruct((n_tok, DIM), jnp.float32),
                   jax.ShapeDtypeStruct((n_tok, NUM_EXPERTS), jnp.float32)],
        compiler_params=pltpu.CompilerParams(
            dimension_semantics=("parallel",)),
    )(xc, win_e, wout_e, vecs_e, gw, gb, acat, bcat)

    return (pred_z.reshape(b, s, d),
            gate_probs.reshape(b, s, NUM_EXPERTS),
            z_target.reshape(b, s, d))


# speedup vs baseline: 1.8905x; 1.0505x over previous
import functools

import jax
import jax.numpy as jnp
from jax import lax
from jax.experimental import pallas as pl
from jax.experimental.pallas import tpu as pltpu

DIM = 1024
LAYERS = 3
NUM_EXPERTS = 4
TOP_K = 2
RANK = 16
TB = 512


def _layernorm(y):
    mu = jnp.mean(y, axis=-1, keepdims=True)
    var = jnp.mean(y * y, axis=-1, keepdims=True) - mu * mu
    return (y - mu) * lax.rsqrt(var + 1e-5)


def _liquid_layers(x, win_ref, wout_ref, vec_ref, cast=None):
    mm = lambda a, w: lax.dot_general(
        a if cast is None else a.astype(cast), w,
        (((1,), (1,)), ((), ())), preferred_element_type=jnp.float32)
    for l in range(LAYERS):
        bin_ = vec_ref[5 * l + 0 : 5 * l + 1, :]
        dec = vec_ref[5 * l + 1 : 5 * l + 2, :]
        bout = vec_ref[5 * l + 2 : 5 * l + 3, :]
        gam = vec_ref[5 * l + 3 : 5 * l + 4, :]
        bet = vec_ref[5 * l + 4 : 5 * l + 5, :]
        g = jax.nn.sigmoid(mm(x, win_ref[l]) + bin_)
        ns = g * (x * dec)
        out = mm(ns, wout_ref[l]) + bout
        x = _layernorm(out + x) * gam + bet
    return x


def _tgt_body(x_ref, win_ref, wout_ref, vec_ref, z_ref):
    z_ref[...] = _liquid_layers(x_ref[...], win_ref, wout_ref, vec_ref,
                                cast=jnp.bfloat16)


def _ctx_body(x_ref, win_ref, wout_ref, vec_ref, gw_ref, gb_ref,
              acat_ref, bcat_ref, pred_ref, probs_ref):
    z = _liquid_layers(x_ref[...], win_ref, wout_ref, vec_ref)
    logits = lax.dot_general(z, gw_ref[...], (((1,), (1,)), ((), ())),
                             preferred_element_type=jnp.float32) + gb_ref[...]
    m = jnp.max(logits, axis=-1, keepdims=True)
    e = jnp.exp(logits - m)
    probs_ref[...] = e / jnp.sum(e, axis=-1, keepdims=True)
    ii = lax.broadcasted_iota(jnp.int32, (TB, NUM_EXPERTS), 1)
    cnt = jnp.zeros((TB, NUM_EXPERTS), jnp.int32)
    for j in range(NUM_EXPERTS):
        lj = logits[:, j : j + 1]
        cnt = cnt + ((lj > logits) | ((lj == logits) & (j < ii))).astype(jnp.int32)
    mask = (cnt < TOP_K).astype(jnp.float32)
    h = lax.dot_general(z, acat_ref[...], (((1,), (1,)), ((), ())),
                        preferred_element_type=jnp.float32)
    h = 0.5 * h * (1.0 + lax.erf(h * 0.7071067811865476))
    mask64 = jnp.concatenate(
        [jnp.broadcast_to(mask[:, i : i + 1], (TB, RANK))
         for i in range(NUM_EXPERTS)], axis=1)
    pred_ref[...] = lax.dot_general(h * mask64, bcat_ref[...],
                                    (((1,), (0,)), ((), ())),
                                    preferred_element_type=jnp.float32)


def _stack_params(blocks):
    win = jnp.stack([p['win'] for p in blocks])
    wout = jnp.stack([p['wout'] for p in blocks])
    vecs = jnp.stack([v for p in blocks
                      for v in (p['bin'], p['decay'], p['bout'],
                                p['gamma'], p['beta'])])
    return win, wout, vecs


def kernel(x_context, x_target, params):
    b, s, d = x_context.shape
    n_tok = b * s
    grid = (n_tok // TB,)
    xc = x_context.reshape(n_tok, d)
    xt = x_target.reshape(n_tok, d)

    win_e, wout_e, vecs_e = _stack_params(params['encoder'])
    win_t, wout_t, vecs_t = _stack_params(params['target_encoder'])
    win_t = win_t.astype(jnp.bfloat16)
    wout_t = wout_t.astype(jnp.bfloat16)
    pred = params['predictor']
    gw = pred['gate_w']
    gb = pred['gate_b'].reshape(1, NUM_EXPERTS)
    acat = jnp.concatenate([e['A'] for e in pred['experts']], axis=0)
    bcat = jnp.concatenate([e['B'].T for e in pred['experts']], axis=0)

    tok_spec = pl.BlockSpec((TB, DIM), lambda i: (i, 0))
    full = lambda shape: pl.BlockSpec(shape, lambda i: (0,) * len(shape))

    z_target = pl.pallas_call(
        _tgt_body,
        grid=grid,
        in_specs=[tok_spec,
                  full((LAYERS, DIM, DIM)), full((LAYERS, DIM, DIM)),
                  full((5 * LAYERS, DIM))],
        out_specs=tok_spec,
        out_shape=jax.ShapeDtypeStruct((n_tok, DIM), jnp.float32),
        compiler_params=pltpu.CompilerParams(
            dimension_semantics=("parallel",)),
    )(xt, win_t, wout_t, vecs_t)

    pred_z, gate_probs = pl.pallas_call(
        _ctx_body,
        grid=grid,
        in_specs=[tok_spec,
                  full((LAYERS, DIM, DIM)), full((LAYERS, DIM, DIM)),
                  full((5 * LAYERS, DIM)),
                  full((NUM_EXPERTS, DIM)), full((1, NUM_EXPERTS)),
                  full((NUM_EXPERTS * RANK, DIM)), full((NUM_EXPERTS * RANK, DIM))],
        out_specs=[tok_spec, pl.BlockSpec((TB, NUM_EXPERTS), lambda i: (i, 0))],
        out_shape=[jax.ShapeDtypeStruct((n_tok, DIM), jnp.float32),
                   jax.ShapeDtypeStruct((n_tok, NUM_EXPERTS), jnp.float32)],
        compiler_params=pltpu.CompilerParams(
            dimension_semantics=("parallel",)),
    )(xc, win_e, wout_e, vecs_e, gw, gb, acat, bcat)

    return (pred_z.reshape(b, s, d),
            gate_probs.reshape(b, s, NUM_EXPERTS),
            z_target.reshape(b, s, d))
